# Initial kernel scaffold; baseline (speedup 1.0000x reference)
import jax
import jax.numpy as jnp
from jax.experimental import pallas as pl


def kernel(questions, word_embedding):
    raise NotImplementedError("write your pallas kernel here")


# speedup vs baseline: 2.4837x; 2.4837x over previous
import functools

import numpy as np
import jax
import jax.numpy as jnp
from jax import lax
from jax.experimental import pallas as pl
from jax.experimental.pallas import tpu as pltpu
from jax.experimental.pallas import tpu_sc as plsc

_VOCAB = 1000000
_EMBED = 64
_BATCH = 16384
_SLEN = 50

_NC = 2
_NS = 16
_NW = _NC * _NS

_CB = 16
_IDX_MINOR = 100
_ROWS_PER_STEP = _CB * _SLEN
_IDX_ROWS = _ROWS_PER_STEP // _IDX_MINOR
_BPW = _BATCH // _NW
_STEPS = _BPW // _CB


def _pe_weights():
    l = np.array([[1.0 - s / _SLEN - e / _EMBED * (1.0 - 2.0 * s / _SLEN)
                   for s in range(_SLEN)] for e in range(_EMBED)],
                 dtype=np.float32)
    return jnp.asarray(l.reshape(_SLEN, _EMBED))


def kernel(questions, word_embedding):
    q2 = questions.reshape(_BATCH * _SLEN // _IDX_MINOR, _IDX_MINOR)
    w = _pe_weights()
    mesh = plsc.VectorSubcoreMesh(core_axis_name="c", subcore_axis_name="s")

    @functools.partial(
        pl.kernel,
        out_type=jax.ShapeDtypeStruct((_BATCH, _EMBED), jnp.float32),
        mesh=mesh,
        scratch_types=[
            pltpu.VMEM((_IDX_ROWS, _IDX_MINOR), jnp.int32),
            pltpu.VMEM((_ROWS_PER_STEP, _EMBED), jnp.float32),
            pltpu.VMEM((_SLEN, _EMBED), jnp.float32),
            pltpu.VMEM((_CB, _EMBED), jnp.float32),
            pltpu.SemaphoreType.DMA,
        ],
        compiler_params=pltpu.CompilerParams(use_tc_tiling_on_sc=False),
    )
    def run(q_hbm, t_hbm, w_hbm, o_hbm, idx_v, rows_v, w_v, out_v, sem):
        wid = lax.axis_index("s") * _NC + lax.axis_index("c")
        pltpu.sync_copy(w_hbm, w_v)

        @pl.loop(0, _STEPS)
        def _(s):
            qrow0 = wid * (_BPW * _SLEN // _IDX_MINOR) + s * _IDX_ROWS
            pltpu.sync_copy(q_hbm.at[pl.ds(qrow0, _IDX_ROWS)], idx_v)
            cps = [
                pltpu.async_copy(
                    t_hbm.at[idx_v.at[j]],
                    rows_v.at[pl.ds(j * _IDX_MINOR, _IDX_MINOR)],
                    sem,
                )
                for j in range(_IDX_ROWS)
            ]
            for cp in cps:
                cp.wait()

            for b0 in range(0, _CB, 4):
                def body(l, accs):
                    ws = [w_v[l, pl.ds(16 * k, 16)] for k in range(4)]
                    nxt = []
                    for g in range(4):
                        r = (b0 + g) * _SLEN + l
                        for k in range(4):
                            nxt.append(accs[g * 4 + k]
                                       + rows_v[r, pl.ds(16 * k, 16)] * ws[k])
                    return tuple(nxt)

                zero = jnp.zeros((16,), jnp.float32)
                accs = lax.fori_loop(0, _SLEN, body, (zero,) * 16)
                for g in range(4):
                    for k in range(4):
                        out_v[b0 + g, pl.ds(16 * k, 16)] = accs[g * 4 + k]

            pltpu.sync_copy(out_v, o_hbm.at[pl.ds(wid * _BPW + s * _CB, _CB)])

    return run(q2, word_embedding, w)


# speedup vs baseline: 2.7373x; 1.1021x over previous
import functools

import numpy as np
import jax
import jax.numpy as jnp
from jax import lax
from jax.experimental import pallas as pl
from jax.experimental.pallas import tpu as pltpu
from jax.experimental.pallas import tpu_sc as plsc

_VOCAB = 1000000
_EMBED = 64
_BATCH = 16384
_SLEN = 50

_NC = 2
_NS = 16
_NW = _NC * _NS

_CB = 16
_IDX_MINOR = 100
_ROWS_PER_STEP = _CB * _SLEN
_IDX_ROWS = _ROWS_PER_STEP // _IDX_MINOR
_BPW = _BATCH // _NW
_STEPS = _BPW // _CB


def _pe_weights():
    l = np.array([[1.0 - s / _SLEN - e / _EMBED * (1.0 - 2.0 * s / _SLEN)
                   for s in range(_SLEN)] for e in range(_EMBED)],
                 dtype=np.float32)
    return jnp.asarray(l.reshape(_SLEN, _EMBED))


def kernel(questions, word_embedding):
    q2 = questions.reshape(_BATCH * _SLEN // _IDX_MINOR, _IDX_MINOR)
    w = _pe_weights()
    mesh = plsc.VectorSubcoreMesh(core_axis_name="c", subcore_axis_name="s")

    @functools.partial(
        pl.kernel,
        out_type=jax.ShapeDtypeStruct((_BATCH, _EMBED), jnp.float32),
        mesh=mesh,
        scratch_types=[
            pltpu.VMEM((_IDX_ROWS, _IDX_MINOR), jnp.int32),
            pltpu.VMEM((_IDX_ROWS, _IDX_MINOR), jnp.int32),
            pltpu.VMEM((_ROWS_PER_STEP, _EMBED), jnp.float32),
            pltpu.VMEM((_ROWS_PER_STEP, _EMBED), jnp.float32),
            pltpu.VMEM((_SLEN, _EMBED), jnp.float32),
            pltpu.VMEM((_CB, _EMBED), jnp.float32),
            pltpu.SemaphoreType.DMA,
            pltpu.SemaphoreType.DMA,
        ],
        compiler_params=pltpu.CompilerParams(use_tc_tiling_on_sc=False),
    )
    def run(q_hbm, t_hbm, w_hbm, o_hbm,
            idx0, idx1, rows0, rows1, w_v, out_v, sem0, sem1):
        wid = lax.axis_index("s") * _NC + lax.axis_index("c")
        qbase = wid * (_BPW * _SLEN // _IDX_MINOR)
        pltpu.sync_copy(w_hbm, w_v)

        def fire(step, idx_v, rows_v, sem):
            pltpu.sync_copy(q_hbm.at[pl.ds(qbase + step * _IDX_ROWS, _IDX_ROWS)],
                            idx_v)
            for j in range(_IDX_ROWS):
                pltpu.async_copy(
                    t_hbm.at[idx_v.at[j]],
                    rows_v.at[pl.ds(j * _IDX_MINOR, _IDX_MINOR)],
                    sem,
                )

        def drain(idx_v, rows_v, sem):
            for j in range(_IDX_ROWS):
                pltpu.make_async_copy(
                    t_hbm.at[idx_v.at[j]],
                    rows_v.at[pl.ds(j * _IDX_MINOR, _IDX_MINOR)],
                    sem,
                ).wait()

        def compute(step, rows_v):
            for b0 in range(0, _CB, 4):
                def body(l, accs):
                    ws = [w_v[l, pl.ds(16 * k, 16)] for k in range(4)]
                    nxt = []
                    for g in range(4):
                        r = (b0 + g) * _SLEN + l
                        for k in range(4):
                            nxt.append(accs[g * 4 + k]
                                       + rows_v[r, pl.ds(16 * k, 16)] * ws[k])
                    return tuple(nxt)

                zero = jnp.zeros((16,), jnp.float32)
                accs = lax.fori_loop(0, _SLEN, body, (zero,) * 16)
                for g in range(4):
                    for k in range(4):
                        out_v[b0 + g, pl.ds(16 * k, 16)] = accs[g * 4 + k]
            pltpu.sync_copy(out_v,
                            o_hbm.at[pl.ds(wid * _BPW + step * _CB, _CB)])

        fire(jnp.int32(0), idx0, rows0, sem0)

        @pl.loop(0, _STEPS, step=2)
        def _(s0):
            fire(s0 + 1, idx1, rows1, sem1)
            drain(idx0, rows0, sem0)
            compute(s0, rows0)

            @pl.when(s0 + 2 < _STEPS)
            def _():
                fire(s0 + 2, idx0, rows0, sem0)

            drain(idx1, rows1, sem1)
            compute(s0 + 1, rows1)

    return run(q2, word_embedding, w)
